# baseline (speedup 1.0000x reference)
import jax
import jax.numpy as jnp
from jax.experimental import pallas as pl

_TOPK = 256


def _topk_mask_kernel(x_ref, o_ref):
    x = x_ref[...]
    R, C = x.shape
    u = jax.lax.bitcast_convert_type(x, jnp.uint32)
    s = u >> jnp.uint32(31)
    flip = jnp.where(s == 0, jnp.uint32(0x80000000), jnp.uint32(0xFFFFFFFF))
    m = u ^ flip

    def cond(carry):
        i, _, sep, _ = carry
        return (i < 32) & (jnp.min(sep) == 0)

    def body(carry):
        i, prefix, sep, septhr = carry
        b = jnp.uint32(31) - i.astype(jnp.uint32)
        cand = prefix | jnp.left_shift(jnp.uint32(1), b)
        cnt = jnp.sum((m >= cand[:, None]).astype(jnp.int32), axis=1)
        newly = (sep == 0) & (cnt == _TOPK)
        septhr = jnp.where(newly, cand, septhr)
        sep = jnp.where(newly, jnp.int32(1), sep)
        prefix = jnp.where(cnt >= _TOPK, cand, prefix)
        return i + 1, prefix, sep, septhr

    carry0 = (jnp.int32(0),
              jnp.zeros((R,), jnp.uint32),
              jnp.zeros((R,), jnp.int32),
              jnp.zeros((R,), jnp.uint32))
    _, prefix, sep, septhr = jax.lax.while_loop(cond, body, carry0)

    def fast_path(_):
        return jnp.where(m >= septhr[:, None], x, jnp.float32(0.0))

    def tie_path(_):
        gt = m > prefix[:, None]
        eq_f = (m == prefix[:, None]).astype(jnp.float32)
        need = (jnp.float32(_TOPK)
                - jnp.sum(gt.astype(jnp.float32), axis=1))

        nchunk = C // 128
        e3 = eq_f.reshape(R * nchunk, 128)
        tri128 = (jax.lax.broadcasted_iota(jnp.int32, (128, 128), 0)
                  <= jax.lax.broadcasted_iota(jnp.int32, (128, 128), 1)
                  ).astype(jnp.float32)
        pref_in = jnp.dot(e3, tri128,
                          preferred_element_type=jnp.float32)
        pref_in = pref_in.reshape(R, nchunk, 128)
        chunk_tot = eq_f.reshape(R, nchunk, 128).sum(axis=2)
        trin = (jax.lax.broadcasted_iota(jnp.int32, (nchunk, nchunk), 0)
                < jax.lax.broadcasted_iota(jnp.int32, (nchunk, nchunk), 1)
                ).astype(jnp.float32)
        chunk_excl = jnp.dot(chunk_tot, trin,
                             preferred_element_type=jnp.float32)
        rank = (pref_in + chunk_excl[:, :, None]).reshape(R, C)
        keep_eq = (eq_f > 0) & (rank <= need[:, None])
        out_exact = jnp.where(gt | keep_eq, x, jnp.float32(0.0))
        out_sep = jnp.where(m >= septhr[:, None], x, jnp.float32(0.0))
        return jnp.where((sep > 0)[:, None], out_sep, out_exact)

    o_ref[...] = jax.lax.cond(jnp.min(sep) > 0, fast_path, tie_path,
                              operand=None)


@jax.jit
def kernel(x):
    R, C = x.shape
    BR = 16
    return pl.pallas_call(
        _topk_mask_kernel,
        grid=(R // BR,),
        in_specs=[pl.BlockSpec((BR, C), lambda i: (i, 0))],
        out_specs=pl.BlockSpec((BR, C), lambda i: (i, 0)),
        out_shape=jax.ShapeDtypeStruct((R, C), x.dtype),
    )(x)
